# baseline (speedup 1.0000x reference)
import jax, jax.numpy as jnp
import numpy as np
from jax import lax
from jax.experimental import pallas as pl
from jax.experimental.pallas import tpu as pltpu

_SIZE = 100000
_SMOOTHING = 0.1
_PAD_ID = 3

_EPS = np.float32(_SMOOTHING / (_SIZE - 2))
_TGT_COEFF = float(_EPS - np.float32(1.0 - _SMOOTHING))
_ROW_CONST = float(
    (_SIZE - 2) * (_EPS * np.log(_EPS))
    + np.float32(1.0 - _SMOOTHING) * np.log(np.float32(1.0 - _SMOOTHING))
)

_BR = 16


def _kl_kernel(t_ref, x_ref, out_ref):
    j = pl.program_id(0)

    t = t_ref[:, :]
    x = x_ref[:, :]
    row_ok = t != _PAD_ID

    rs = jnp.sum(x, axis=1, keepdims=True)
    main = -_EPS * jnp.sum(jnp.where(row_ok, rs, jnp.float32(0.0)))
    corr3 = _EPS * jnp.sum(
        jnp.where(row_ok, x[:, _PAD_ID : _PAD_ID + 1], jnp.float32(0.0))
    )
    count = jnp.sum(row_ok.astype(jnp.float32))

    lane = lax.broadcasted_iota(jnp.int32, (1, 128), 1)
    gacc = jnp.zeros((1, 128), jnp.float32)
    for r in range(_BR):
        idx = t_ref[r, 0]
        base = pl.multiple_of((idx // 128) * 128, 128)
        win = x_ref[r : r + 1, pl.ds(base, 128)]
        idx_sel = jnp.where(idx != _PAD_ID, idx - base, jnp.int32(-1))
        gacc = gacc + jnp.where(lane == idx_sel, win, jnp.float32(0.0))
    g = jnp.sum(gacc)

    contrib = main + corr3 + jnp.float32(_ROW_CONST) * count + _TGT_COEFF * g

    @pl.when(j == 0)
    def _init():
        out_ref[:, :] = jnp.zeros((1, 1), jnp.float32)

    out_ref[:, :] += contrib.reshape(1, 1)


@jax.jit
def _run(x, t):
    n = x.shape[0]
    out = pl.pallas_call(
        _kl_kernel,
        grid=(n // _BR,),
        in_specs=[
            pl.BlockSpec((_BR, 1), lambda j: (j, 0)),
            pl.BlockSpec((_BR, _SIZE), lambda j: (j, 0)),
        ],
        out_specs=pl.BlockSpec((1, 1), lambda j: (0, 0)),
        out_shape=jax.ShapeDtypeStruct((1, 1), jnp.float32),
    )(t, x)
    return out[0, 0]


def kernel(x, target, nwords):
    x2 = x.reshape(-1, _SIZE)
    t = target.reshape(-1).astype(jnp.int32)[:, None]
    return _run(x2, t) / nwords


# speedup vs baseline: 1.0347x; 1.0347x over previous
import jax, jax.numpy as jnp
import numpy as np
from jax import lax
from jax.experimental import pallas as pl
from jax.experimental.pallas import tpu as pltpu

_SIZE = 100000
_SMOOTHING = 0.1
_PAD_ID = 3

_EPS = np.float32(_SMOOTHING / (_SIZE - 2))
_TGT_COEFF = float(_EPS - np.float32(1.0 - _SMOOTHING))
_ROW_CONST = float(
    (_SIZE - 2) * (_EPS * np.log(_EPS))
    + np.float32(1.0 - _SMOOTHING) * np.log(np.float32(1.0 - _SMOOTHING))
)

_BR = 32


def _kl_kernel(t_ref, x_ref, out_ref):
    j = pl.program_id(0)

    t = t_ref[:, :]
    x = x_ref[:, :]
    row_ok = t != _PAD_ID

    rs = jnp.sum(x, axis=1, keepdims=True)
    main = -_EPS * jnp.sum(jnp.where(row_ok, rs, jnp.float32(0.0)))
    corr3 = _EPS * jnp.sum(
        jnp.where(row_ok, x[:, _PAD_ID : _PAD_ID + 1], jnp.float32(0.0))
    )
    count = jnp.sum(row_ok.astype(jnp.float32))

    lane = lax.broadcasted_iota(jnp.int32, (1, 128), 1)
    gacc = jnp.zeros((1, 128), jnp.float32)
    for r in range(_BR):
        idx = t_ref[r, 0]
        base = pl.multiple_of((idx // 128) * 128, 128)
        win = x_ref[r : r + 1, pl.ds(base, 128)]
        idx_sel = jnp.where(idx != _PAD_ID, idx - base, jnp.int32(-1))
        gacc = gacc + jnp.where(lane == idx_sel, win, jnp.float32(0.0))
    g = jnp.sum(gacc)

    contrib = main + corr3 + jnp.float32(_ROW_CONST) * count + _TGT_COEFF * g

    @pl.when(j == 0)
    def _init():
        out_ref[:, :] = jnp.zeros((1, 1), jnp.float32)

    out_ref[:, :] += contrib.reshape(1, 1)


@jax.jit
def _run(x, t):
    n = x.shape[0]
    out = pl.pallas_call(
        _kl_kernel,
        grid=(n // _BR,),
        in_specs=[
            pl.BlockSpec((_BR, 1), lambda j: (j, 0)),
            pl.BlockSpec((_BR, _SIZE), lambda j: (j, 0)),
        ],
        out_specs=pl.BlockSpec((1, 1), lambda j: (0, 0)),
        out_shape=jax.ShapeDtypeStruct((1, 1), jnp.float32),
    )(t, x)
    return out[0, 0]


def kernel(x, target, nwords):
    x2 = x.reshape(-1, _SIZE)
    t = target.reshape(-1).astype(jnp.int32)[:, None]
    return _run(x2, t) / nwords


# speedup vs baseline: 1.0351x; 1.0004x over previous
import jax, jax.numpy as jnp
import numpy as np
from jax import lax
from jax.experimental import pallas as pl

_SIZE = 100000
_SMOOTHING = 0.1
_PAD_ID = 3

_EPS = np.float32(_SMOOTHING / (_SIZE - 2))
_TGT_COEFF = float(_EPS - np.float32(1.0 - _SMOOTHING))
_ROW_CONST = float(
    (_SIZE - 2) * (_EPS * np.log(_EPS))
    + np.float32(1.0 - _SMOOTHING) * np.log(np.float32(1.0 - _SMOOTHING))
)

_BR = 32


def _kl_kernel(t_ref, x_ref, out_ref):
    j = pl.program_id(0)

    t = t_ref[:, :]
    x = x_ref[:, :]
    row_ok = t != _PAD_ID

    rs = jnp.sum(x, axis=1, keepdims=True)
    main = -_EPS * jnp.sum(jnp.where(row_ok, rs, jnp.float32(0.0)))
    corr3 = _EPS * jnp.sum(
        jnp.where(row_ok, x[:, _PAD_ID : _PAD_ID + 1], jnp.float32(0.0))
    )
    count = jnp.sum(row_ok.astype(jnp.float32))

    _W = 160
    lane = lax.broadcasted_iota(jnp.int32, (1, _W), 1)
    gacc = jnp.zeros((1, _W), jnp.float32)
    for r in range(_BR):
        idx = t_ref[r, 0]
        base = pl.multiple_of(
            jnp.minimum((idx // 128) * 128, _SIZE - _W), 128
        )
        win = x_ref[r : r + 1, pl.ds(base, _W)]
        idx_sel = jnp.where(idx != _PAD_ID, idx - base, jnp.int32(-1))
        gacc = gacc + jnp.where(lane == idx_sel, win, jnp.float32(0.0))
    g = jnp.sum(gacc)

    contrib = main + corr3 + jnp.float32(_ROW_CONST) * count + _TGT_COEFF * g

    @pl.when(j == 0)
    def _init():
        out_ref[:, :] = jnp.zeros((1, 1), jnp.float32)

    out_ref[:, :] += contrib.reshape(1, 1)


@jax.jit
def _run(x, t):
    n = x.shape[0]
    out = pl.pallas_call(
        _kl_kernel,
        grid=(n // _BR,),
        in_specs=[
            pl.BlockSpec((_BR, 1), lambda j: (j, 0)),
            pl.BlockSpec((_BR, _SIZE), lambda j: (j, 0)),
        ],
        out_specs=pl.BlockSpec((1, 1), lambda j: (0, 0)),
        out_shape=jax.ShapeDtypeStruct((1, 1), jnp.float32),
    )(t, x)
    return out[0, 0]


def kernel(x, target, nwords):
    x2 = x.reshape(-1, _SIZE)
    t = target.reshape(-1).astype(jnp.int32)[:, None]
    return _run(x2, t) / nwords
